# baseline (speedup 1.0000x reference)
import functools

import jax
import jax.numpy as jnp
from jax import lax
from jax.experimental import pallas as pl
from jax.experimental.pallas import tpu as pltpu
from jax.experimental.pallas import tpu_sc as plsc

N_USERS = 25000
N_ITEMS = 25000
D = 64
DW = D // 2
BATCH = 2048

NC = 2
NS = 16
L = 16
CHUNK = 64
GR = 6
SR = 3
LAG_G = 3
LAG_S = 1
G = 9
N_PAD = 25088
SLAB = N_PAD // NS
WB = 32

MASK_HI = -65536
ROUND = 0x8000


def _to_packed(x):
    xb = x.astype(jnp.bfloat16)
    lo = jnp.concatenate([xb[:, 0:16], xb[:, 32:48]], axis=1)
    hi = jnp.concatenate([xb[:, 16:32], xb[:, 48:64]], axis=1)
    return jax.lax.bitcast_convert_type(
        jnp.stack([lo, hi], axis=-1), jnp.int32)


def _unpack_words(w):
    lo = plsc.bitcast(lax.shift_left(w, 16), jnp.float32)
    hi = plsc.bitcast(lax.bitwise_and(w, MASK_HI), jnp.float32)
    return lo, hi


def _pack_words(lo, hi):
    li = lax.shift_right_logical(plsc.bitcast(lo, jnp.int32) + ROUND, 16)
    hic = lax.bitwise_and(plsc.bitcast(hi, jnp.int32) + ROUND, MASK_HI)
    return lax.bitwise_or(li, hic)


def _propagate(zeros, tbl, edges, nct):
    mesh = plsc.VectorSubcoreMesh(core_axis_name="c", subcore_axis_name="s")

    @functools.partial(
        pl.kernel,
        out_type=jax.ShapeDtypeStruct((2, N_PAD, DW), jnp.int32),
        mesh=mesh,
        scratch_types=(
            [pltpu.VMEM((G, 3, CHUNK), jnp.int32)] * 2
            + [pltpu.VMEM((CHUNK, DW), jnp.int32)] * GR
            + [pltpu.VMEM((CHUNK, D), jnp.float32)] * SR
            + [pltpu.VMEM_SHARED((N_PAD, D), jnp.float32)]
            + [pltpu.SemaphoreType.DMA] * (GR + SR + 2)
        ),
        compiler_params=pltpu.CompilerParams(needs_layout_passes=False,
                                             use_tc_tiling_on_sc=False),
    )
    def layer(zeros_hbm, tbl_hbm, edg_hbm, out, ibA, ibB, *rest):
        gbufs = rest[:GR]
        sbufs = rest[GR:GR + SR]
        acc = rest[GR + SR]
        gsems = rest[GR + SR + 1:2 * GR + SR + 1]
        ssems = rest[2 * GR + SR + 1:2 * GR + 2 * SR + 1]
        isems = rest[2 * GR + 2 * SR + 1:2 * GR + 2 * SR + 3]
        cid = lax.axis_index("c")
        sid = lax.axis_index("s")
        ibs = (ibA, ibB)

        pltpu.sync_copy(zeros_hbm, acc.at[pl.ds(sid * SLAB, SLAB)])
        plsc.subcore_barrier()

        ngr = nct // G
        nbody = nct // (2 * G)

        def side():
            dr = cid
            sr = 1 - cid
            table = tbl_hbm.at[sr]

            def sel(t):
                tm = t % (2 * G)
                return (0 if tm < G else 1), t % G

            def g_start(t):
                tb, u = sel(t)
                k = t % GR
                pltpu.async_copy(table.at[ibs[tb].at[u, sr]], gbufs[k],
                                 gsems[k])

            def g_wait(t):
                tb, u = sel(t)
                k = t % GR
                pltpu.make_async_copy(table.at[ibs[tb].at[u, sr]], gbufs[k],
                                      gsems[k]).wait()

            def s_start(t):
                tb, u = sel(t)
                m = t % SR
                pltpu.async_copy(sbufs[m], acc.at[ibs[tb].at[u, dr]],
                                 ssems[m], add=True)

            def s_wait(t):
                tb, u = sel(t)
                m = t % SR
                pltpu.make_async_copy(sbufs[m], acc.at[ibs[tb].at[u, dr]],
                                      ssems[m]).wait()

            def i_start(tb, g):
                pltpu.async_copy(edg_hbm.at[sid, pl.ds(g * G, G)], ibs[tb],
                                 isems[tb])

            def i_wait(tb):
                pltpu.make_async_copy(edg_hbm.at[sid, pl.ds(0, G)], ibs[tb],
                                      isems[tb]).wait()

            def scale(t):
                tb, u = sel(t)
                gbuf = gbufs[t % GR]
                sbuf = sbufs[t % SR]

                def body(e, _):
                    vv = plsc.bitcast(
                        plsc.load_gather(
                            ibs[tb], [jnp.full((L,), u, jnp.int32),
                                      jnp.full((L,), 2, jnp.int32),
                                      jnp.full((L,), e, jnp.int32)]),
                        jnp.float32)
                    for h in range(2):
                        w = gbuf[e, pl.ds(h * L, L)]
                        lo, hi = _unpack_words(w)
                        sbuf[e, pl.ds(h * 2 * L, L)] = lo * vv
                        sbuf[e, pl.ds(h * 2 * L + L, L)] = hi * vv
                    return 0

                lax.fori_loop(0, CHUNK, body, 0, unroll=8)

            pltpu.sync_copy(edg_hbm.at[sid, pl.ds(0, G)], ibA)
            for t in range(LAG_G):
                g_start(t)

            def body(j, _):
                for t in range(2 * G):
                    g_wait(t)
                    if t == LAG_S:
                        i_start(1, 2 * j + 1)
                    if t == G + LAG_S:
                        @pl.when(2 * j + 2 < ngr)
                        def _():
                            i_start(0, 2 * j + 2)
                    if t == G - LAG_G:
                        i_wait(1)
                    if t == 2 * G - LAG_G:
                        @pl.when(2 * j + 2 < ngr)
                        def _():
                            i_wait(0)
                    if t < 2 * G - LAG_G:
                        g_start(t + LAG_G)
                    else:
                        @pl.when(2 * j + 2 < ngr)
                        def _():
                            g_start(t + LAG_G)
                    scale(t)
                    s_start(t)
                    if t < LAG_S:
                        @pl.when(j > 0)
                        def _():
                            s_wait(t - LAG_S)
                    else:
                        s_wait(t - LAG_S)
                return 0

            lax.fori_loop(0, nbody, body, 0)
            for t in range(LAG_S):
                s_wait(2 * G - LAG_S + t)

        side()
        plsc.subcore_barrier()

        fbuf = sbufs[0]
        bbuf = gbufs[0]

        def blk(b, _):
            row0 = sid * SLAB + b * WB
            pltpu.sync_copy(acc.at[pl.ds(row0, WB)], fbuf.at[pl.ds(0, WB)])

            def rowfn(e, _):
                for h in range(2):
                    lo = fbuf[e, pl.ds(h * 2 * L, L)]
                    hi = fbuf[e, pl.ds(h * 2 * L + L, L)]
                    bbuf[e, pl.ds(h * L, L)] = _pack_words(lo, hi)
                return 0

            lax.fori_loop(0, WB, rowfn, 0, unroll=4)
            pltpu.sync_copy(bbuf.at[pl.ds(0, WB)], out.at[cid, pl.ds(row0, WB)])
            return 0

        lax.fori_loop(0, SLAB // WB, blk, 0)

    return layer(zeros, tbl, edges)


def _finalize(t0, t1, t2, idx2):
    rows = BATCH // NS
    mesh = plsc.VectorSubcoreMesh(core_axis_name="c", subcore_axis_name="s")

    @functools.partial(
        pl.kernel,
        out_type=jax.ShapeDtypeStruct((2, BATCH, D), jnp.float32),
        mesh=mesh,
        scratch_types=[
            pltpu.VMEM((rows,), jnp.int32),
            pltpu.VMEM((rows, D), jnp.float32),
            pltpu.VMEM((rows, DW), jnp.int32),
            pltpu.VMEM((rows, DW), jnp.int32),
            pltpu.SemaphoreType.DMA,
        ],
        compiler_params=pltpu.CompilerParams(needs_layout_passes=False,
                                             use_tc_tiling_on_sc=False),
    )
    def fin(t0_hbm, t1_hbm, t2_hbm, idx_hbm,
            out, idx_v, g0, g1, g2, sem):
        cid = lax.axis_index("c")
        sid = lax.axis_index("s")
        base = sid * rows

        def side():
            pltpu.sync_copy(idx_hbm.at[cid, pl.ds(base, rows)], idx_v)
            pltpu.async_copy(t0_hbm.at[cid].at[idx_v], g0, sem).wait()
            pltpu.async_copy(t1_hbm.at[cid].at[idx_v], g1, sem).wait()
            pltpu.async_copy(t2_hbm.at[cid].at[idx_v], g2, sem).wait()

            def mean_body(e, _):
                for h in range(2):
                    lo1, hi1 = _unpack_words(g1[e, pl.ds(h * L, L)])
                    lo2, hi2 = _unpack_words(g2[e, pl.ds(h * L, L)])
                    sl_lo = pl.ds(h * 2 * L, L)
                    sl_hi = pl.ds(h * 2 * L + L, L)
                    g0[e, sl_lo] = (g0[e, sl_lo] + lo1 + lo2) * (1.0 / 3.0)
                    g0[e, sl_hi] = (g0[e, sl_hi] + hi1 + hi2) * (1.0 / 3.0)
                return 0

            lax.fori_loop(0, rows, mean_body, 0, unroll=4)
            pltpu.sync_copy(g0, out.at[cid, pl.ds(base, rows)])

        side()

    return fin(t0, t1, t2, idx2)


def kernel(users, pos_items, user_emb, item_emb, adj_row, adj_col, adj_val):
    E = adj_row.shape[0] // 2
    r = adj_row[:E].astype(jnp.int32)
    ci = adj_col[:E].astype(jnp.int32) - N_USERS
    val = adj_val[:E]

    group = NS * CHUNK * 2 * G
    e_pad = ((E + group - 1) // group) * group
    pad = e_pad - E
    if pad:
        r = jnp.concatenate([r, jnp.zeros((pad,), jnp.int32)])
        ci = jnp.concatenate([ci, jnp.zeros((pad,), jnp.int32)])
        val = jnp.concatenate([val, jnp.zeros((pad,), jnp.float32)])
    nct = e_pad // (NS * CHUNK)
    edges = jnp.stack(
        [r.reshape(NS, nct, CHUNK), ci.reshape(NS, nct, CHUNK),
         jax.lax.bitcast_convert_type(val, jnp.int32).reshape(NS, nct, CHUNK)],
        axis=2)

    t0p = jnp.stack([_to_packed(user_emb), _to_packed(item_emb)])
    t0f = jnp.stack([user_emb, item_emb])
    idx2 = jnp.stack([users.astype(jnp.int32), pos_items.astype(jnp.int32)])
    zeros = jnp.zeros((SLAB, D), jnp.float32)
    t1 = _propagate(zeros, t0p, edges, nct)
    t2 = _propagate(zeros, t1, edges, nct)
    fo = _finalize(t0f, t1, t2, idx2)
    return fo[0], fo[1]


# speedup vs baseline: 1.0601x; 1.0601x over previous
import functools

import jax
import jax.numpy as jnp
from jax import lax
from jax.experimental import pallas as pl
from jax.experimental.pallas import tpu as pltpu
from jax.experimental.pallas import tpu_sc as plsc

N_USERS = 25000
N_ITEMS = 25000
D = 64
DW = D // 2
BATCH = 2048

NC = 2
NS = 16
L = 16
CHUNK = 64
GR = 6
SR = 3
LAG_G = 3
LAG_S = 1
G = 9
N_PAD = 25088
SLAB = N_PAD // NS
WB = 32

MASK_HI = -65536
ROUND = 0x8000


def _to_packed(x):
    xb = x.astype(jnp.bfloat16)
    lo = jnp.concatenate([xb[:, 0:16], xb[:, 32:48]], axis=1)
    hi = jnp.concatenate([xb[:, 16:32], xb[:, 48:64]], axis=1)
    return jax.lax.bitcast_convert_type(
        jnp.stack([lo, hi], axis=-1), jnp.int32)


def _unpack_words(w):
    lo = plsc.bitcast(lax.shift_left(w, 16), jnp.float32)
    hi = plsc.bitcast(lax.bitwise_and(w, MASK_HI), jnp.float32)
    return lo, hi


def _pack_words(lo, hi):
    li = lax.shift_right_logical(plsc.bitcast(lo, jnp.int32) + ROUND, 16)
    hic = lax.bitwise_and(plsc.bitcast(hi, jnp.int32) + ROUND, MASK_HI)
    return lax.bitwise_or(li, hic)


def _propagate(zeros, tbl, edges, nct):
    mesh = plsc.VectorSubcoreMesh(core_axis_name="c", subcore_axis_name="s")

    @functools.partial(
        pl.kernel,
        out_type=jax.ShapeDtypeStruct((2, N_PAD, DW), jnp.int32),
        mesh=mesh,
        scratch_types=(
            [pltpu.VMEM((G, 3, CHUNK), jnp.int32)] * 2
            + [pltpu.VMEM((CHUNK, DW), jnp.int32)] * GR
            + [pltpu.VMEM((CHUNK, D), jnp.float32)] * SR
            + [pltpu.VMEM_SHARED((N_PAD, D), jnp.float32)]
            + [pltpu.SemaphoreType.DMA] * (GR + SR + 2)
        ),
        compiler_params=pltpu.CompilerParams(needs_layout_passes=False,
                                             use_tc_tiling_on_sc=False),
    )
    def layer(zeros_hbm, tbl_hbm, edg_hbm, out, ibA, ibB, *rest):
        gbufs = rest[:GR]
        sbufs = rest[GR:GR + SR]
        acc = rest[GR + SR]
        gsems = rest[GR + SR + 1:2 * GR + SR + 1]
        ssems = rest[2 * GR + SR + 1:2 * GR + 2 * SR + 1]
        isems = rest[2 * GR + 2 * SR + 1:2 * GR + 2 * SR + 3]
        cid = lax.axis_index("c")
        sid = lax.axis_index("s")
        ibs = (ibA, ibB)

        pltpu.sync_copy(zeros_hbm, acc.at[pl.ds(sid * SLAB, SLAB)])
        plsc.subcore_barrier()

        ngr = nct // G
        nbody = nct // (2 * G)

        def side():
            dr = cid
            sr = 1 - cid
            table = tbl_hbm.at[sr]

            def sel(t):
                tm = t % (2 * G)
                return (0 if tm < G else 1), t % G

            def g_start(t):
                tb, u = sel(t)
                k = t % GR
                pltpu.async_copy(table.at[ibs[tb].at[u, sr]], gbufs[k],
                                 gsems[k])

            def g_wait(t):
                tb, u = sel(t)
                k = t % GR
                pltpu.make_async_copy(table.at[ibs[tb].at[u, sr]], gbufs[k],
                                      gsems[k]).wait()

            def s_start(t):
                tb, u = sel(t)
                m = t % SR
                pltpu.async_copy(sbufs[m], acc.at[ibs[tb].at[u, dr]],
                                 ssems[m], add=True)

            def s_wait(t):
                tb, u = sel(t)
                m = t % SR
                pltpu.make_async_copy(sbufs[m], acc.at[ibs[tb].at[u, dr]],
                                      ssems[m]).wait()

            def i_start(tb, g):
                pltpu.async_copy(edg_hbm.at[sid, pl.ds(g * G, G)], ibs[tb],
                                 isems[tb])

            def i_wait(tb):
                pltpu.make_async_copy(edg_hbm.at[sid, pl.ds(0, G)], ibs[tb],
                                      isems[tb]).wait()

            def scale(t):
                tb, u = sel(t)
                gbuf = gbufs[t % GR]
                sbuf = sbufs[t % SR]

                def body(e, _):
                    vv = plsc.bitcast(
                        plsc.load_gather(
                            ibs[tb], [jnp.full((L,), u, jnp.int32),
                                      jnp.full((L,), 2, jnp.int32),
                                      jnp.full((L,), e, jnp.int32)]),
                        jnp.float32)
                    for h in range(2):
                        w = gbuf[e, pl.ds(h * L, L)]
                        lo, hi = _unpack_words(w)
                        sbuf[e, pl.ds(h * 2 * L, L)] = lo * vv
                        sbuf[e, pl.ds(h * 2 * L + L, L)] = hi * vv
                    return 0

                lax.fori_loop(0, CHUNK, body, 0, unroll=4)

            pltpu.sync_copy(edg_hbm.at[sid, pl.ds(0, G)], ibA)
            for t in range(LAG_G):
                g_start(t)

            def body(j, _):
                for t in range(2 * G):
                    g_wait(t)
                    scale(t)
                    s_start(t)
                    if t == LAG_S:
                        i_start(1, 2 * j + 1)
                    if t == G + LAG_S:
                        @pl.when(2 * j + 2 < ngr)
                        def _():
                            i_start(0, 2 * j + 2)
                    if t < LAG_S:
                        @pl.when(j > 0)
                        def _():
                            s_wait(t - LAG_S)
                    else:
                        s_wait(t - LAG_S)
                    if t == G - LAG_G:
                        i_wait(1)
                    if t == 2 * G - LAG_G:
                        @pl.when(2 * j + 2 < ngr)
                        def _():
                            i_wait(0)
                    if t < 2 * G - LAG_G:
                        g_start(t + LAG_G)
                    else:
                        @pl.when(2 * j + 2 < ngr)
                        def _():
                            g_start(t + LAG_G)
                return 0

            lax.fori_loop(0, nbody, body, 0)
            for t in range(LAG_S):
                s_wait(2 * G - LAG_S + t)

        side()
        plsc.subcore_barrier()

        fbuf = sbufs[0]
        bbuf = gbufs[0]

        def blk(b, _):
            row0 = sid * SLAB + b * WB
            pltpu.sync_copy(acc.at[pl.ds(row0, WB)], fbuf.at[pl.ds(0, WB)])

            def rowfn(e, _):
                for h in range(2):
                    lo = fbuf[e, pl.ds(h * 2 * L, L)]
                    hi = fbuf[e, pl.ds(h * 2 * L + L, L)]
                    bbuf[e, pl.ds(h * L, L)] = _pack_words(lo, hi)
                return 0

            lax.fori_loop(0, WB, rowfn, 0, unroll=4)
            pltpu.sync_copy(bbuf.at[pl.ds(0, WB)], out.at[cid, pl.ds(row0, WB)])
            return 0

        lax.fori_loop(0, SLAB // WB, blk, 0)

    return layer(zeros, tbl, edges)


def _finalize(t0, t1, t2, idx2):
    rows = BATCH // NS
    mesh = plsc.VectorSubcoreMesh(core_axis_name="c", subcore_axis_name="s")

    @functools.partial(
        pl.kernel,
        out_type=jax.ShapeDtypeStruct((2, BATCH, D), jnp.float32),
        mesh=mesh,
        scratch_types=[
            pltpu.VMEM((rows,), jnp.int32),
            pltpu.VMEM((rows, D), jnp.float32),
            pltpu.VMEM((rows, DW), jnp.int32),
            pltpu.VMEM((rows, DW), jnp.int32),
            pltpu.SemaphoreType.DMA,
        ],
        compiler_params=pltpu.CompilerParams(needs_layout_passes=False,
                                             use_tc_tiling_on_sc=False),
    )
    def fin(t0_hbm, t1_hbm, t2_hbm, idx_hbm,
            out, idx_v, g0, g1, g2, sem):
        cid = lax.axis_index("c")
        sid = lax.axis_index("s")
        base = sid * rows

        def side():
            pltpu.sync_copy(idx_hbm.at[cid, pl.ds(base, rows)], idx_v)
            pltpu.async_copy(t0_hbm.at[cid].at[idx_v], g0, sem).wait()
            pltpu.async_copy(t1_hbm.at[cid].at[idx_v], g1, sem).wait()
            pltpu.async_copy(t2_hbm.at[cid].at[idx_v], g2, sem).wait()

            def mean_body(e, _):
                for h in range(2):
                    lo1, hi1 = _unpack_words(g1[e, pl.ds(h * L, L)])
                    lo2, hi2 = _unpack_words(g2[e, pl.ds(h * L, L)])
                    sl_lo = pl.ds(h * 2 * L, L)
                    sl_hi = pl.ds(h * 2 * L + L, L)
                    g0[e, sl_lo] = (g0[e, sl_lo] + lo1 + lo2) * (1.0 / 3.0)
                    g0[e, sl_hi] = (g0[e, sl_hi] + hi1 + hi2) * (1.0 / 3.0)
                return 0

            lax.fori_loop(0, rows, mean_body, 0, unroll=4)
            pltpu.sync_copy(g0, out.at[cid, pl.ds(base, rows)])

        side()

    return fin(t0, t1, t2, idx2)


def kernel(users, pos_items, user_emb, item_emb, adj_row, adj_col, adj_val):
    E = adj_row.shape[0] // 2
    r = adj_row[:E].astype(jnp.int32)
    ci = adj_col[:E].astype(jnp.int32) - N_USERS
    val = adj_val[:E]

    group = NS * CHUNK * 2 * G
    e_pad = ((E + group - 1) // group) * group
    pad = e_pad - E
    if pad:
        r = jnp.concatenate([r, jnp.zeros((pad,), jnp.int32)])
        ci = jnp.concatenate([ci, jnp.zeros((pad,), jnp.int32)])
        val = jnp.concatenate([val, jnp.zeros((pad,), jnp.float32)])
    nct = e_pad // (NS * CHUNK)
    edges = jnp.stack(
        [r.reshape(NS, nct, CHUNK), ci.reshape(NS, nct, CHUNK),
         jax.lax.bitcast_convert_type(val, jnp.int32).reshape(NS, nct, CHUNK)],
        axis=2)

    t0p = jnp.stack([_to_packed(user_emb), _to_packed(item_emb)])
    t0f = jnp.stack([user_emb, item_emb])
    idx2 = jnp.stack([users.astype(jnp.int32), pos_items.astype(jnp.int32)])
    zeros = jnp.zeros((SLAB, D), jnp.float32)
    t1 = _propagate(zeros, t0p, edges, nct)
    t2 = _propagate(zeros, t1, edges, nct)
    fo = _finalize(t0f, t1, t2, idx2)
    return fo[0], fo[1]
